# baseline (speedup 1.0000x reference)
import functools

import jax
import jax.numpy as jnp
from jax import lax
from jax.experimental import pallas as pl
from jax.experimental.pallas import tpu as pltpu
from jax.experimental.pallas import tpu_sc as plsc

N = 10000
E = 320000
D = 128
R = 4

NC = 2
NS = 16
NW = NC * NS

CHUNK = 80
CPT0 = 184
CPT1 = 66
E0 = NS * CPT0 * CHUNK
N_PAD = 10112
ROWS_PT = N_PAD // NS
ROW_CHUNKS = [(0, 128), (128, 128), (256, 128), (384, 128), (512, 120)]

BN = 1000
NB = N // BN
NROWS = E // CHUNK


def _table_body(x_ref, w_ref, o_ref):
    o_ref[...] = jnp.dot(x_ref[...], w_ref[0], preferred_element_type=jnp.float32)


def _make_table(x, W_rel):
    bn = 1000
    nb = N // bn
    return pl.pallas_call(
        _table_body,
        grid=(nb, R),
        in_specs=[
            pl.BlockSpec((bn, D), lambda i, r: (i, 0)),
            pl.BlockSpec((1, D, D), lambda i, r: (r, 0, 0)),
        ],
        out_specs=pl.BlockSpec((bn, D), lambda i, r: (r * nb + i, 0)),
        out_shape=jax.ShapeDtypeStruct((R * N, D), jnp.float32),
    )(x, W_rel)


def _gidx_body(src_ref, et_ref, o_ref):
    o_ref[...] = et_ref[...] * N + src_ref[: E // 128]


def _make_gidx(eidx2d, et2d):
    nrows = E // 128
    return pl.pallas_call(
        _gidx_body,
        grid=(1,),
        in_specs=[
            pl.BlockSpec((2 * nrows, 128), lambda i: (0, 0)),
            pl.BlockSpec((nrows, 128), lambda i: (0, 0)),
        ],
        out_specs=pl.BlockSpec((nrows, 128), lambda i: (0, 0)),
        out_shape=jax.ShapeDtypeStruct((nrows, 128), jnp.int32),
    )(eidx2d, et2d)


def _sc_agg_body(table_hbm, gidx_hbm, didx_hbm, zrow_hbm,
                 agg_out,
                 gidx_v, didx_v, rows0_v, rows1_v,
                 agg_sh, sem0, sem1):
    c = lax.axis_index("c")
    s = lax.axis_index("s")
    w = c * NS + s
    base = s * ROWS_PT

    for off, sz in ROW_CHUNKS:
        pltpu.sync_copy(zrow_hbm.at[pl.ds(0, sz)], agg_sh.at[pl.ds(base + off, sz)])

    @pl.when(c == 0)
    def _():
        row = s * CPT0
        pltpu.sync_copy(gidx_hbm.at[pl.ds(row, CPT0)], gidx_v)
        pltpu.sync_copy(didx_hbm.at[pl.ds(NROWS + row, CPT0)], didx_v)

    @pl.when(c != 0)
    def _():
        row = NS * CPT0 + s * CPT1
        pltpu.sync_copy(gidx_hbm.at[pl.ds(row, CPT1)], gidx_v.at[pl.ds(0, CPT1)])
        pltpu.sync_copy(didx_hbm.at[pl.ds(NROWS + row, CPT1)],
                        didx_v.at[pl.ds(0, CPT1)])

    plsc.subcore_barrier()

    bufs = (rows0_v, rows1_v)
    sems = (sem0, sem1)
    cpt = lax.select(c == 0, CPT0, CPT1)

    pltpu.async_copy(table_hbm.at[gidx_v.at[0]], rows0_v, sem0)
    pltpu.async_copy(table_hbm.at[gidx_v.at[1]], rows1_v, sem1)

    def body(j2, carry):
        for k in range(2):
            j = j2 * 2 + k
            buf, sm = bufs[k], sems[k]
            pltpu.make_async_copy(table_hbm.at[gidx_v.at[j]], buf, sm).wait()
            pltpu.sync_copy(buf, agg_sh.at[didx_v.at[j]], add=True)

            @pl.when(j + 2 < carry)
            def _():
                pltpu.async_copy(table_hbm.at[gidx_v.at[j + 2]], buf, sm)
        return carry

    lax.fori_loop(0, cpt // 2, body, cpt)

    plsc.subcore_barrier()

    for off, sz in ROW_CHUNKS:
        pltpu.sync_copy(agg_sh.at[pl.ds(base + off, sz)], agg_out.at[c, pl.ds(base + off, sz)])


def _sc_aggregate(table, gidx3, didx3, zrow):
    mesh = plsc.VectorSubcoreMesh(core_axis_name="c", subcore_axis_name="s",
                                  num_cores=NC, num_subcores=NS)
    fn = functools.partial(
        pl.kernel,
        out_type=jax.ShapeDtypeStruct((NC, N_PAD, D), jnp.float32),
        mesh=mesh,
        scratch_types=[
            pltpu.VMEM((CPT0, CHUNK), jnp.int32),
            pltpu.VMEM((CPT0, CHUNK), jnp.int32),
            pltpu.VMEM((CHUNK, D), jnp.float32),
            pltpu.VMEM((CHUNK, D), jnp.float32),
            pltpu.VMEM_SHARED((N_PAD, D), jnp.float32),
            pltpu.SemaphoreType.DMA,
            pltpu.SemaphoreType.DMA,
        ],
        compiler_params=pltpu.CompilerParams(use_tc_tiling_on_sc=False),
    )(_sc_agg_body)
    return fn(table, gidx3, didx3, zrow)


def _sc_deg_body(didx_hbm, ones_hbm, z16_hbm,
                 deg_out,
                 didx_v, ones_v,
                 deg_sh):
    c = lax.axis_index("c")
    s = lax.axis_index("s")
    w = c * NS + s
    base = s * ROWS_PT

    pltpu.sync_copy(ones_hbm, ones_v)
    for off, sz in ROW_CHUNKS:
        pltpu.sync_copy(z16_hbm.at[pl.ds(0, sz)], deg_sh.at[pl.ds(base + off, sz)])

    @pl.when(c == 0)
    def _():
        pltpu.sync_copy(didx_hbm.at[pl.ds(NROWS + s * CPT0, CPT0)], didx_v)

    @pl.when(c != 0)
    def _():
        pltpu.sync_copy(didx_hbm.at[pl.ds(NROWS + NS * CPT0 + s * CPT1, CPT1)],
                        didx_v.at[pl.ds(0, CPT1)])

    plsc.subcore_barrier()

    def body(j, carry):
        pltpu.sync_copy(ones_v, deg_sh.at[didx_v.at[j]], add=True)
        return carry

    lax.fori_loop(0, lax.select(c == 0, CPT0, CPT1), body, 0)

    plsc.subcore_barrier()

    for off, sz in ROW_CHUNKS:
        pltpu.sync_copy(deg_sh.at[pl.ds(base + off, sz)], deg_out.at[c, pl.ds(base + off, sz)])


def _sc_degrees(didx3, ones16, z16):
    mesh = plsc.VectorSubcoreMesh(core_axis_name="c", subcore_axis_name="s",
                                  num_cores=NC, num_subcores=NS)
    fn = functools.partial(
        pl.kernel,
        out_type=jax.ShapeDtypeStruct((NC, N_PAD, 16), jnp.float32),
        mesh=mesh,
        scratch_types=[
            pltpu.VMEM((CPT0, CHUNK), jnp.int32),
            pltpu.VMEM((CHUNK, 16), jnp.float32),
            pltpu.VMEM_SHARED((N_PAD, 16), jnp.float32),
        ],
        compiler_params=pltpu.CompilerParams(use_tc_tiling_on_sc=False),
    )(_sc_deg_body)
    return fn(didx3, ones16, z16)


def _h_body(aggp_ref, degp_ref, x_ref, ws_ref, b_ref, h_ref, stats_ref, acc_ref):
    i = pl.program_id(0)

    agg = aggp_ref[0] + aggp_ref[1]
    deg = degp_ref[0, :, 0:1] + degp_ref[1, :, 0:1]
    deg = jnp.maximum(deg, 1.0)
    self_part = jnp.dot(x_ref[...], ws_ref[...], preferred_element_type=jnp.float32)
    h = jnp.maximum(agg / deg + self_part + b_ref[...], 0.0)
    h_ref[...] = h

    rm = jnp.sum(h, axis=1, keepdims=True) * (1.0 / D)
    diff = h - rm
    rv = jnp.sum(diff * diff, axis=1, keepdims=True) * (1.0 / D)
    rs = jnp.sqrt(rv)

    pm = jnp.sum(rm)
    pm2 = jnp.sum(rm * rm)
    ps = jnp.sum(rs)
    ps2 = jnp.sum(rs * rs)

    @pl.when(i == 0)
    def _():
        acc_ref[0] = 0.0
        acc_ref[1] = 0.0
        acc_ref[2] = 0.0
        acc_ref[3] = 0.0

    acc_ref[0] += pm
    acc_ref[1] += pm2
    acc_ref[2] += ps
    acc_ref[3] += ps2

    @pl.when(i == NB - 1)
    def _():
        inv_n = 1.0 / N
        mu_m = acc_ref[0] * inv_n
        var_m = jnp.maximum(acc_ref[1] * inv_n - mu_m * mu_m, 0.0)
        mu_s = acc_ref[2] * inv_n
        var_s = jnp.maximum(acc_ref[3] * inv_n - mu_s * mu_s, 0.0)
        std_mu = jnp.sqrt(var_m)
        std_var = jnp.sqrt(var_s)
        row = lax.broadcasted_iota(jnp.int32, (8, 128), 0)
        stats_ref[...] = jnp.where(row == 0, std_mu, std_var)


def _make_h(aggp, degp, x, W_self, b2):
    return pl.pallas_call(
        _h_body,
        grid=(NB,),
        in_specs=[
            pl.BlockSpec((2, BN, D), lambda i: (0, i, 0)),
            pl.BlockSpec((2, BN, 16), lambda i: (0, i, 0)),
            pl.BlockSpec((BN, D), lambda i: (i, 0)),
            pl.BlockSpec((D, D), lambda i: (0, 0)),
            pl.BlockSpec((1, D), lambda i: (0, 0)),
        ],
        out_specs=[
            pl.BlockSpec((BN, D), lambda i: (i, 0)),
            pl.BlockSpec((8, 128), lambda i: (0, 0)),
        ],
        out_shape=[
            jax.ShapeDtypeStruct((N, D), jnp.float32),
            jax.ShapeDtypeStruct((8, 128), jnp.float32),
        ],
        scratch_shapes=[pltpu.SMEM((4,), jnp.float32)],
    )(aggp, degp, x, W_self, b2)


def _z_body(h_ref, stats_ref, eb_ref, eg_ref, z_ref):
    h = h_ref[...]
    rm = jnp.sum(h, axis=1, keepdims=True) * (1.0 / D)
    diff = h - rm
    rv = jnp.sum(diff * diff, axis=1, keepdims=True) * (1.0 / D)
    rs = jnp.sqrt(rv)
    std_mu = stats_ref[0, 0]
    std_var = stats_ref[1, 0]
    beta = rm + eb_ref[...] * std_mu
    gam = rs + eg_ref[...] * std_var
    z_ref[...] = beta + gam * (diff / (rs + 1e-05))


def _make_z(h, stats, eps_beta, eps_gam):
    return pl.pallas_call(
        _z_body,
        grid=(NB,),
        in_specs=[
            pl.BlockSpec((BN, D), lambda i: (i, 0)),
            pl.BlockSpec((8, 128), lambda i: (0, 0)),
            pl.BlockSpec((BN, 1), lambda i: (i, 0)),
            pl.BlockSpec((BN, 1), lambda i: (i, 0)),
        ],
        out_specs=pl.BlockSpec((BN, D), lambda i: (i, 0)),
        out_shape=jax.ShapeDtypeStruct((N, D), jnp.float32),
    )(h, stats, eps_beta, eps_gam)


def kernel(x, edge_index, edge_type, W_rel, W_self, b, eps_beta, eps_gam):
    eidx128 = edge_index.reshape(2 * (E // 128), 128)
    eidx80 = edge_index.reshape(2 * NROWS, CHUNK)

    table = _make_table(x, W_rel)
    gidx = _make_gidx(eidx128, edge_type.reshape(E // 128, 128))
    gidx3 = gidx.reshape(NROWS, CHUNK)
    didx3 = eidx80

    zrow = jnp.zeros((128, D), jnp.float32)
    ones16 = jnp.ones((CHUNK, 16), jnp.float32)
    z16 = jnp.zeros((128, 16), jnp.float32)

    aggp = _sc_aggregate(table, gidx3, didx3, zrow)
    degp = _sc_degrees(didx3, ones16, z16)

    b2 = b.reshape(1, D)
    h, stats = _make_h(aggp, degp, x, W_self, b2)
    z = _make_z(h, stats, eps_beta, eps_gam)
    return (h, z)


# speedup vs baseline: 1.0364x; 1.0364x over previous
import functools

import jax
import jax.numpy as jnp
from jax import lax
from jax.experimental import pallas as pl
from jax.experimental.pallas import tpu as pltpu
from jax.experimental.pallas import tpu_sc as plsc

N = 10000
E = 320000
D = 128
R = 4

NC = 2
NS = 16
NW = NC * NS

CHUNK = 80
CPT0 = 172
CPT1 = 78
E0 = NS * CPT0 * CHUNK
N_PAD = 10112
ROWS_PT = N_PAD // NS
ROW_CHUNKS = [(0, 128), (128, 128), (256, 128), (384, 128), (512, 120)]

BN = 1000
NB = N // BN
NROWS = E // CHUNK


def _table_body(x_ref, w_ref, o_ref):
    o_ref[...] = jnp.dot(x_ref[...], w_ref[0], preferred_element_type=jnp.float32)


def _make_table(x, W_rel):
    bn = 1000
    nb = N // bn
    return pl.pallas_call(
        _table_body,
        grid=(nb, R),
        in_specs=[
            pl.BlockSpec((bn, D), lambda i, r: (i, 0)),
            pl.BlockSpec((1, D, D), lambda i, r: (r, 0, 0)),
        ],
        out_specs=pl.BlockSpec((bn, D), lambda i, r: (r * nb + i, 0)),
        out_shape=jax.ShapeDtypeStruct((R * N, D), jnp.float32),
    )(x, W_rel)


def _gidx_body(src_ref, et_ref, o_ref):
    o_ref[...] = et_ref[...] * N + src_ref[: E // 128]


def _make_gidx(eidx2d, et2d):
    nrows = E // 128
    return pl.pallas_call(
        _gidx_body,
        grid=(1,),
        in_specs=[
            pl.BlockSpec((2 * nrows, 128), lambda i: (0, 0)),
            pl.BlockSpec((nrows, 128), lambda i: (0, 0)),
        ],
        out_specs=pl.BlockSpec((nrows, 128), lambda i: (0, 0)),
        out_shape=jax.ShapeDtypeStruct((nrows, 128), jnp.int32),
    )(eidx2d, et2d)


def _sc_agg_body(table_hbm, gidx_hbm, didx_hbm, zrow_hbm,
                 agg_out,
                 gidx_v, didx_v, rows0_v, rows1_v,
                 agg_sh, sem0, sem1):
    c = lax.axis_index("c")
    s = lax.axis_index("s")
    w = c * NS + s
    base = s * ROWS_PT

    for off, sz in ROW_CHUNKS:
        pltpu.sync_copy(zrow_hbm.at[pl.ds(0, sz)], agg_sh.at[pl.ds(base + off, sz)])

    @pl.when(c == 0)
    def _():
        row = s * CPT0
        pltpu.sync_copy(gidx_hbm.at[pl.ds(row, CPT0)], gidx_v)
        pltpu.sync_copy(didx_hbm.at[pl.ds(NROWS + row, CPT0)], didx_v)

    @pl.when(c != 0)
    def _():
        row = NS * CPT0 + s * CPT1
        pltpu.sync_copy(gidx_hbm.at[pl.ds(row, CPT1)], gidx_v.at[pl.ds(0, CPT1)])
        pltpu.sync_copy(didx_hbm.at[pl.ds(NROWS + row, CPT1)],
                        didx_v.at[pl.ds(0, CPT1)])

    plsc.subcore_barrier()

    bufs = (rows0_v, rows1_v)
    sems = (sem0, sem1)
    cpt = lax.select(c == 0, CPT0, CPT1)

    pltpu.async_copy(table_hbm.at[gidx_v.at[0]], rows0_v, sem0)
    pltpu.async_copy(table_hbm.at[gidx_v.at[1]], rows1_v, sem1)

    def body(j2, carry):
        for k in range(2):
            j = j2 * 2 + k
            buf, sm = bufs[k], sems[k]
            pltpu.make_async_copy(table_hbm.at[gidx_v.at[j]], buf, sm).wait()
            pltpu.sync_copy(buf, agg_sh.at[didx_v.at[j]], add=True)

            @pl.when(j + 2 < carry)
            def _():
                pltpu.async_copy(table_hbm.at[gidx_v.at[j + 2]], buf, sm)
        return carry

    lax.fori_loop(0, cpt // 2, body, cpt)

    plsc.subcore_barrier()

    for off, sz in ROW_CHUNKS:
        pltpu.sync_copy(agg_sh.at[pl.ds(base + off, sz)], agg_out.at[c, pl.ds(base + off, sz)])


def _sc_aggregate(table, gidx3, didx3, zrow):
    mesh = plsc.VectorSubcoreMesh(core_axis_name="c", subcore_axis_name="s",
                                  num_cores=NC, num_subcores=NS)
    fn = functools.partial(
        pl.kernel,
        out_type=jax.ShapeDtypeStruct((NC, N_PAD, D), jnp.float32),
        mesh=mesh,
        scratch_types=[
            pltpu.VMEM((CPT0, CHUNK), jnp.int32),
            pltpu.VMEM((CPT0, CHUNK), jnp.int32),
            pltpu.VMEM((CHUNK, D), jnp.float32),
            pltpu.VMEM((CHUNK, D), jnp.float32),
            pltpu.VMEM_SHARED((N_PAD, D), jnp.float32),
            pltpu.SemaphoreType.DMA,
            pltpu.SemaphoreType.DMA,
        ],
        compiler_params=pltpu.CompilerParams(use_tc_tiling_on_sc=False),
    )(_sc_agg_body)
    return fn(table, gidx3, didx3, zrow)


def _sc_deg_body(didx_hbm, ones_hbm, z16_hbm,
                 deg_out,
                 didx_v, ones_v,
                 deg_sh):
    c = lax.axis_index("c")
    s = lax.axis_index("s")
    w = c * NS + s
    base = s * ROWS_PT

    pltpu.sync_copy(ones_hbm, ones_v)
    for off, sz in ROW_CHUNKS:
        pltpu.sync_copy(z16_hbm.at[pl.ds(0, sz)], deg_sh.at[pl.ds(base + off, sz)])

    @pl.when(c == 0)
    def _():
        pltpu.sync_copy(didx_hbm.at[pl.ds(NROWS + s * CPT0, CPT0)], didx_v)

    @pl.when(c != 0)
    def _():
        pltpu.sync_copy(didx_hbm.at[pl.ds(NROWS + NS * CPT0 + s * CPT1, CPT1)],
                        didx_v.at[pl.ds(0, CPT1)])

    plsc.subcore_barrier()

    def body(j, carry):
        pltpu.sync_copy(ones_v, deg_sh.at[didx_v.at[j]], add=True)
        return carry

    lax.fori_loop(0, lax.select(c == 0, CPT0, CPT1), body, 0)

    plsc.subcore_barrier()

    for off, sz in ROW_CHUNKS:
        pltpu.sync_copy(deg_sh.at[pl.ds(base + off, sz)], deg_out.at[c, pl.ds(base + off, sz)])


def _sc_degrees(didx3, ones16, z16):
    mesh = plsc.VectorSubcoreMesh(core_axis_name="c", subcore_axis_name="s",
                                  num_cores=NC, num_subcores=NS)
    fn = functools.partial(
        pl.kernel,
        out_type=jax.ShapeDtypeStruct((NC, N_PAD, 16), jnp.float32),
        mesh=mesh,
        scratch_types=[
            pltpu.VMEM((CPT0, CHUNK), jnp.int32),
            pltpu.VMEM((CHUNK, 16), jnp.float32),
            pltpu.VMEM_SHARED((N_PAD, 16), jnp.float32),
        ],
        compiler_params=pltpu.CompilerParams(use_tc_tiling_on_sc=False),
    )(_sc_deg_body)
    return fn(didx3, ones16, z16)


def _h_body(aggp_ref, degp_ref, x_ref, ws_ref, b_ref, h_ref, stats_ref, acc_ref):
    i = pl.program_id(0)

    agg = aggp_ref[0] + aggp_ref[1]
    deg = degp_ref[0, :, 0:1] + degp_ref[1, :, 0:1]
    deg = jnp.maximum(deg, 1.0)
    self_part = jnp.dot(x_ref[...], ws_ref[...], preferred_element_type=jnp.float32)
    h = jnp.maximum(agg / deg + self_part + b_ref[...], 0.0)
    h_ref[...] = h

    rm = jnp.sum(h, axis=1, keepdims=True) * (1.0 / D)
    diff = h - rm
    rv = jnp.sum(diff * diff, axis=1, keepdims=True) * (1.0 / D)
    rs = jnp.sqrt(rv)

    pm = jnp.sum(rm)
    pm2 = jnp.sum(rm * rm)
    ps = jnp.sum(rs)
    ps2 = jnp.sum(rs * rs)

    @pl.when(i == 0)
    def _():
        acc_ref[0] = 0.0
        acc_ref[1] = 0.0
        acc_ref[2] = 0.0
        acc_ref[3] = 0.0

    acc_ref[0] += pm
    acc_ref[1] += pm2
    acc_ref[2] += ps
    acc_ref[3] += ps2

    @pl.when(i == NB - 1)
    def _():
        inv_n = 1.0 / N
        mu_m = acc_ref[0] * inv_n
        var_m = jnp.maximum(acc_ref[1] * inv_n - mu_m * mu_m, 0.0)
        mu_s = acc_ref[2] * inv_n
        var_s = jnp.maximum(acc_ref[3] * inv_n - mu_s * mu_s, 0.0)
        std_mu = jnp.sqrt(var_m)
        std_var = jnp.sqrt(var_s)
        row = lax.broadcasted_iota(jnp.int32, (8, 128), 0)
        stats_ref[...] = jnp.where(row == 0, std_mu, std_var)


def _make_h(aggp, degp, x, W_self, b2):
    return pl.pallas_call(
        _h_body,
        grid=(NB,),
        in_specs=[
            pl.BlockSpec((2, BN, D), lambda i: (0, i, 0)),
            pl.BlockSpec((2, BN, 16), lambda i: (0, i, 0)),
            pl.BlockSpec((BN, D), lambda i: (i, 0)),
            pl.BlockSpec((D, D), lambda i: (0, 0)),
            pl.BlockSpec((1, D), lambda i: (0, 0)),
        ],
        out_specs=[
            pl.BlockSpec((BN, D), lambda i: (i, 0)),
            pl.BlockSpec((8, 128), lambda i: (0, 0)),
        ],
        out_shape=[
            jax.ShapeDtypeStruct((N, D), jnp.float32),
            jax.ShapeDtypeStruct((8, 128), jnp.float32),
        ],
        scratch_shapes=[pltpu.SMEM((4,), jnp.float32)],
    )(aggp, degp, x, W_self, b2)


def _z_body(h_ref, stats_ref, eb_ref, eg_ref, z_ref):
    h = h_ref[...]
    rm = jnp.sum(h, axis=1, keepdims=True) * (1.0 / D)
    diff = h - rm
    rv = jnp.sum(diff * diff, axis=1, keepdims=True) * (1.0 / D)
    rs = jnp.sqrt(rv)
    std_mu = stats_ref[0, 0]
    std_var = stats_ref[1, 0]
    beta = rm + eb_ref[...] * std_mu
    gam = rs + eg_ref[...] * std_var
    z_ref[...] = beta + gam * (diff / (rs + 1e-05))


def _make_z(h, stats, eps_beta, eps_gam):
    return pl.pallas_call(
        _z_body,
        grid=(NB,),
        in_specs=[
            pl.BlockSpec((BN, D), lambda i: (i, 0)),
            pl.BlockSpec((8, 128), lambda i: (0, 0)),
            pl.BlockSpec((BN, 1), lambda i: (i, 0)),
            pl.BlockSpec((BN, 1), lambda i: (i, 0)),
        ],
        out_specs=pl.BlockSpec((BN, D), lambda i: (i, 0)),
        out_shape=jax.ShapeDtypeStruct((N, D), jnp.float32),
    )(h, stats, eps_beta, eps_gam)


def kernel(x, edge_index, edge_type, W_rel, W_self, b, eps_beta, eps_gam):
    eidx128 = edge_index.reshape(2 * (E // 128), 128)
    eidx80 = edge_index.reshape(2 * NROWS, CHUNK)

    table = _make_table(x, W_rel)
    gidx = _make_gidx(eidx128, edge_type.reshape(E // 128, 128))
    gidx3 = gidx.reshape(NROWS, CHUNK)
    didx3 = eidx80

    zrow = jnp.zeros((128, D), jnp.float32)
    ones16 = jnp.ones((CHUNK, 16), jnp.float32)
    z16 = jnp.zeros((128, 16), jnp.float32)

    aggp = _sc_aggregate(table, gidx3, didx3, zrow)
    degp = _sc_degrees(didx3, ones16, z16)

    b2 = b.reshape(1, D)
    h, stats = _make_h(aggp, degp, x, W_self, b2)
    z = _make_z(h, stats, eps_beta, eps_gam)
    return (h, z)


# speedup vs baseline: 1.0655x; 1.0281x over previous
import functools

import jax
import jax.numpy as jnp
from jax import lax
from jax.experimental import pallas as pl
from jax.experimental.pallas import tpu as pltpu
from jax.experimental.pallas import tpu_sc as plsc

N = 10000
E = 320000
D = 128
R = 4

NC = 2
NS = 16
NW = NC * NS

CHUNK = 80
CPT0 = 162
CPT1 = 88
E0 = NS * CPT0 * CHUNK
N_PAD = 10112
ROWS_PT = N_PAD // NS
ROW_CHUNKS = [(0, 128), (128, 128), (256, 128), (384, 128), (512, 120)]

BN = 1000
NB = N // BN
NROWS = E // CHUNK


def _table_body(x_ref, w_ref, o_ref):
    o_ref[...] = jnp.dot(x_ref[...], w_ref[0], preferred_element_type=jnp.float32)


def _make_table(x, W_rel):
    bn = 1000
    nb = N // bn
    return pl.pallas_call(
        _table_body,
        grid=(nb, R),
        in_specs=[
            pl.BlockSpec((bn, D), lambda i, r: (i, 0)),
            pl.BlockSpec((1, D, D), lambda i, r: (r, 0, 0)),
        ],
        out_specs=pl.BlockSpec((bn, D), lambda i, r: (r * nb + i, 0)),
        out_shape=jax.ShapeDtypeStruct((R * N, D), jnp.float32),
    )(x, W_rel)


def _gidx_body(src_ref, et_ref, o_ref):
    o_ref[...] = et_ref[...] * N + src_ref[: E // 128]


def _make_gidx(eidx2d, et2d):
    nrows = E // 128
    return pl.pallas_call(
        _gidx_body,
        grid=(1,),
        in_specs=[
            pl.BlockSpec((2 * nrows, 128), lambda i: (0, 0)),
            pl.BlockSpec((nrows, 128), lambda i: (0, 0)),
        ],
        out_specs=pl.BlockSpec((nrows, 128), lambda i: (0, 0)),
        out_shape=jax.ShapeDtypeStruct((nrows, 128), jnp.int32),
    )(eidx2d, et2d)


def _sc_agg_body(table_hbm, gidx_hbm, didx_hbm, zrow_hbm,
                 agg_out,
                 gidx_v, didx_v, rows0_v, rows1_v,
                 agg_sh, sem0, sem1):
    c = lax.axis_index("c")
    s = lax.axis_index("s")
    w = c * NS + s
    base = s * ROWS_PT

    for off, sz in ROW_CHUNKS:
        pltpu.sync_copy(zrow_hbm.at[pl.ds(0, sz)], agg_sh.at[pl.ds(base + off, sz)])

    @pl.when(c == 0)
    def _():
        row = s * CPT0
        pltpu.sync_copy(gidx_hbm.at[pl.ds(row, CPT0)], gidx_v)
        pltpu.sync_copy(didx_hbm.at[pl.ds(NROWS + row, CPT0)], didx_v)

    @pl.when(c != 0)
    def _():
        row = NS * CPT0 + s * CPT1
        pltpu.sync_copy(gidx_hbm.at[pl.ds(row, CPT1)], gidx_v.at[pl.ds(0, CPT1)])
        pltpu.sync_copy(didx_hbm.at[pl.ds(NROWS + row, CPT1)],
                        didx_v.at[pl.ds(0, CPT1)])

    plsc.subcore_barrier()

    bufs = (rows0_v, rows1_v)
    sems = (sem0, sem1)
    cpt = lax.select(c == 0, CPT0, CPT1)

    pltpu.async_copy(table_hbm.at[gidx_v.at[0]], rows0_v, sem0)
    pltpu.async_copy(table_hbm.at[gidx_v.at[1]], rows1_v, sem1)

    def body(j2, carry):
        for k in range(2):
            j = j2 * 2 + k
            buf, sm = bufs[k], sems[k]
            pltpu.make_async_copy(table_hbm.at[gidx_v.at[j]], buf, sm).wait()
            pltpu.sync_copy(buf, agg_sh.at[didx_v.at[j]], add=True)

            @pl.when(j + 2 < carry)
            def _():
                pltpu.async_copy(table_hbm.at[gidx_v.at[j + 2]], buf, sm)
        return carry

    lax.fori_loop(0, cpt // 2, body, cpt)

    plsc.subcore_barrier()

    for off, sz in ROW_CHUNKS:
        pltpu.sync_copy(agg_sh.at[pl.ds(base + off, sz)], agg_out.at[c, pl.ds(base + off, sz)])


def _sc_aggregate(table, gidx3, didx3, zrow):
    mesh = plsc.VectorSubcoreMesh(core_axis_name="c", subcore_axis_name="s",
                                  num_cores=NC, num_subcores=NS)
    fn = functools.partial(
        pl.kernel,
        out_type=jax.ShapeDtypeStruct((NC, N_PAD, D), jnp.float32),
        mesh=mesh,
        scratch_types=[
            pltpu.VMEM((CPT0, CHUNK), jnp.int32),
            pltpu.VMEM((CPT0, CHUNK), jnp.int32),
            pltpu.VMEM((CHUNK, D), jnp.float32),
            pltpu.VMEM((CHUNK, D), jnp.float32),
            pltpu.VMEM_SHARED((N_PAD, D), jnp.float32),
            pltpu.SemaphoreType.DMA,
            pltpu.SemaphoreType.DMA,
        ],
        compiler_params=pltpu.CompilerParams(use_tc_tiling_on_sc=False),
    )(_sc_agg_body)
    return fn(table, gidx3, didx3, zrow)


def _sc_deg_body(didx_hbm, ones_hbm, z16_hbm,
                 deg_out,
                 didx_v, ones_v,
                 deg_sh):
    c = lax.axis_index("c")
    s = lax.axis_index("s")
    w = c * NS + s
    base = s * ROWS_PT

    pltpu.sync_copy(ones_hbm, ones_v)
    for off, sz in ROW_CHUNKS:
        pltpu.sync_copy(z16_hbm.at[pl.ds(0, sz)], deg_sh.at[pl.ds(base + off, sz)])

    @pl.when(c == 0)
    def _():
        pltpu.sync_copy(didx_hbm.at[pl.ds(NROWS + s * CPT0, CPT0)], didx_v)

    @pl.when(c != 0)
    def _():
        pltpu.sync_copy(didx_hbm.at[pl.ds(NROWS + NS * CPT0 + s * CPT1, CPT1)],
                        didx_v.at[pl.ds(0, CPT1)])

    plsc.subcore_barrier()

    def body(j, carry):
        pltpu.sync_copy(ones_v, deg_sh.at[didx_v.at[j]], add=True)
        return carry

    lax.fori_loop(0, lax.select(c == 0, CPT0, CPT1), body, 0)

    plsc.subcore_barrier()

    for off, sz in ROW_CHUNKS:
        pltpu.sync_copy(deg_sh.at[pl.ds(base + off, sz)], deg_out.at[c, pl.ds(base + off, sz)])


def _sc_degrees(didx3, ones16, z16):
    mesh = plsc.VectorSubcoreMesh(core_axis_name="c", subcore_axis_name="s",
                                  num_cores=NC, num_subcores=NS)
    fn = functools.partial(
        pl.kernel,
        out_type=jax.ShapeDtypeStruct((NC, N_PAD, 16), jnp.float32),
        mesh=mesh,
        scratch_types=[
            pltpu.VMEM((CPT0, CHUNK), jnp.int32),
            pltpu.VMEM((CHUNK, 16), jnp.float32),
            pltpu.VMEM_SHARED((N_PAD, 16), jnp.float32),
        ],
        compiler_params=pltpu.CompilerParams(use_tc_tiling_on_sc=False),
    )(_sc_deg_body)
    return fn(didx3, ones16, z16)


def _h_body(aggp_ref, degp_ref, x_ref, ws_ref, b_ref, h_ref, stats_ref, acc_ref):
    i = pl.program_id(0)

    agg = aggp_ref[0] + aggp_ref[1]
    deg = degp_ref[0, :, 0:1] + degp_ref[1, :, 0:1]
    deg = jnp.maximum(deg, 1.0)
    self_part = jnp.dot(x_ref[...], ws_ref[...], preferred_element_type=jnp.float32)
    h = jnp.maximum(agg / deg + self_part + b_ref[...], 0.0)
    h_ref[...] = h

    rm = jnp.sum(h, axis=1, keepdims=True) * (1.0 / D)
    diff = h - rm
    rv = jnp.sum(diff * diff, axis=1, keepdims=True) * (1.0 / D)
    rs = jnp.sqrt(rv)

    pm = jnp.sum(rm)
    pm2 = jnp.sum(rm * rm)
    ps = jnp.sum(rs)
    ps2 = jnp.sum(rs * rs)

    @pl.when(i == 0)
    def _():
        acc_ref[0] = 0.0
        acc_ref[1] = 0.0
        acc_ref[2] = 0.0
        acc_ref[3] = 0.0

    acc_ref[0] += pm
    acc_ref[1] += pm2
    acc_ref[2] += ps
    acc_ref[3] += ps2

    @pl.when(i == NB - 1)
    def _():
        inv_n = 1.0 / N
        mu_m = acc_ref[0] * inv_n
        var_m = jnp.maximum(acc_ref[1] * inv_n - mu_m * mu_m, 0.0)
        mu_s = acc_ref[2] * inv_n
        var_s = jnp.maximum(acc_ref[3] * inv_n - mu_s * mu_s, 0.0)
        std_mu = jnp.sqrt(var_m)
        std_var = jnp.sqrt(var_s)
        row = lax.broadcasted_iota(jnp.int32, (8, 128), 0)
        stats_ref[...] = jnp.where(row == 0, std_mu, std_var)


def _make_h(aggp, degp, x, W_self, b2):
    return pl.pallas_call(
        _h_body,
        grid=(NB,),
        in_specs=[
            pl.BlockSpec((2, BN, D), lambda i: (0, i, 0)),
            pl.BlockSpec((2, BN, 16), lambda i: (0, i, 0)),
            pl.BlockSpec((BN, D), lambda i: (i, 0)),
            pl.BlockSpec((D, D), lambda i: (0, 0)),
            pl.BlockSpec((1, D), lambda i: (0, 0)),
        ],
        out_specs=[
            pl.BlockSpec((BN, D), lambda i: (i, 0)),
            pl.BlockSpec((8, 128), lambda i: (0, 0)),
        ],
        out_shape=[
            jax.ShapeDtypeStruct((N, D), jnp.float32),
            jax.ShapeDtypeStruct((8, 128), jnp.float32),
        ],
        scratch_shapes=[pltpu.SMEM((4,), jnp.float32)],
    )(aggp, degp, x, W_self, b2)


def _z_body(h_ref, stats_ref, eb_ref, eg_ref, z_ref):
    h = h_ref[...]
    rm = jnp.sum(h, axis=1, keepdims=True) * (1.0 / D)
    diff = h - rm
    rv = jnp.sum(diff * diff, axis=1, keepdims=True) * (1.0 / D)
    rs = jnp.sqrt(rv)
    std_mu = stats_ref[0, 0]
    std_var = stats_ref[1, 0]
    beta = rm + eb_ref[...] * std_mu
    gam = rs + eg_ref[...] * std_var
    z_ref[...] = beta + gam * (diff / (rs + 1e-05))


def _make_z(h, stats, eps_beta, eps_gam):
    return pl.pallas_call(
        _z_body,
        grid=(NB,),
        in_specs=[
            pl.BlockSpec((BN, D), lambda i: (i, 0)),
            pl.BlockSpec((8, 128), lambda i: (0, 0)),
            pl.BlockSpec((BN, 1), lambda i: (i, 0)),
            pl.BlockSpec((BN, 1), lambda i: (i, 0)),
        ],
        out_specs=pl.BlockSpec((BN, D), lambda i: (i, 0)),
        out_shape=jax.ShapeDtypeStruct((N, D), jnp.float32),
    )(h, stats, eps_beta, eps_gam)


def kernel(x, edge_index, edge_type, W_rel, W_self, b, eps_beta, eps_gam):
    eidx128 = edge_index.reshape(2 * (E // 128), 128)
    eidx80 = edge_index.reshape(2 * NROWS, CHUNK)

    table = _make_table(x, W_rel)
    gidx = _make_gidx(eidx128, edge_type.reshape(E // 128, 128))
    gidx3 = gidx.reshape(NROWS, CHUNK)
    didx3 = eidx80

    zrow = jnp.zeros((128, D), jnp.float32)
    ones16 = jnp.ones((CHUNK, 16), jnp.float32)
    z16 = jnp.zeros((128, 16), jnp.float32)

    aggp = _sc_aggregate(table, gidx3, didx3, zrow)
    degp = _sc_degrees(didx3, ones16, z16)

    b2 = b.reshape(1, D)
    h, stats = _make_h(aggp, degp, x, W_self, b2)
    z = _make_z(h, stats, eps_beta, eps_gam)
    return (h, z)


# speedup vs baseline: 1.1070x; 1.0389x over previous
import functools

import jax
import jax.numpy as jnp
from jax import lax
from jax.experimental import pallas as pl
from jax.experimental.pallas import tpu as pltpu
from jax.experimental.pallas import tpu_sc as plsc

N = 10000
E = 320000
D = 128
R = 4

NC = 2
NS = 16
NW = NC * NS

CHUNK = 80
CPT0 = 150
CPT1 = 100
E0 = NS * CPT0 * CHUNK
N_PAD = 10112
ROWS_PT = N_PAD // NS
ROW_CHUNKS = [(0, 128), (128, 128), (256, 128), (384, 128), (512, 120)]

BN = 1000
NB = N // BN
NROWS = E // CHUNK


def _table_body(x_ref, w_ref, o_ref):
    o_ref[...] = jnp.dot(x_ref[...], w_ref[0], preferred_element_type=jnp.float32)


def _make_table(x, W_rel):
    bn = 1000
    nb = N // bn
    return pl.pallas_call(
        _table_body,
        grid=(nb, R),
        in_specs=[
            pl.BlockSpec((bn, D), lambda i, r: (i, 0)),
            pl.BlockSpec((1, D, D), lambda i, r: (r, 0, 0)),
        ],
        out_specs=pl.BlockSpec((bn, D), lambda i, r: (r * nb + i, 0)),
        out_shape=jax.ShapeDtypeStruct((R * N, D), jnp.float32),
    )(x, W_rel)


def _gidx_body(src_ref, et_ref, o_ref):
    o_ref[...] = et_ref[...] * N + src_ref[: E // 128]


def _make_gidx(eidx2d, et2d):
    nrows = E // 128
    return pl.pallas_call(
        _gidx_body,
        grid=(1,),
        in_specs=[
            pl.BlockSpec((2 * nrows, 128), lambda i: (0, 0)),
            pl.BlockSpec((nrows, 128), lambda i: (0, 0)),
        ],
        out_specs=pl.BlockSpec((nrows, 128), lambda i: (0, 0)),
        out_shape=jax.ShapeDtypeStruct((nrows, 128), jnp.int32),
    )(eidx2d, et2d)


def _sc_agg_body(table_hbm, gidx_hbm, didx_hbm, zrow_hbm,
                 agg_out,
                 gidx_v, didx_v, rows0_v, rows1_v,
                 agg_sh, sem0, sem1):
    c = lax.axis_index("c")
    s = lax.axis_index("s")
    w = c * NS + s
    base = s * ROWS_PT

    for off, sz in ROW_CHUNKS:
        pltpu.sync_copy(zrow_hbm.at[pl.ds(0, sz)], agg_sh.at[pl.ds(base + off, sz)])

    @pl.when(c == 0)
    def _():
        row = s * CPT0
        pltpu.sync_copy(gidx_hbm.at[pl.ds(row, CPT0)], gidx_v)
        pltpu.sync_copy(didx_hbm.at[pl.ds(NROWS + row, CPT0)], didx_v)

    @pl.when(c != 0)
    def _():
        row = NS * CPT0 + s * CPT1
        pltpu.sync_copy(gidx_hbm.at[pl.ds(row, CPT1)], gidx_v.at[pl.ds(0, CPT1)])
        pltpu.sync_copy(didx_hbm.at[pl.ds(NROWS + row, CPT1)],
                        didx_v.at[pl.ds(0, CPT1)])

    plsc.subcore_barrier()

    bufs = (rows0_v, rows1_v)
    sems = (sem0, sem1)
    cpt = lax.select(c == 0, CPT0, CPT1)

    pltpu.async_copy(table_hbm.at[gidx_v.at[0]], rows0_v, sem0)
    pltpu.async_copy(table_hbm.at[gidx_v.at[1]], rows1_v, sem1)

    def body(j2, carry):
        for k in range(2):
            j = j2 * 2 + k
            buf, sm = bufs[k], sems[k]
            pltpu.make_async_copy(table_hbm.at[gidx_v.at[j]], buf, sm).wait()
            pltpu.sync_copy(buf, agg_sh.at[didx_v.at[j]], add=True)

            @pl.when(j + 2 < carry)
            def _():
                pltpu.async_copy(table_hbm.at[gidx_v.at[j + 2]], buf, sm)
        return carry

    lax.fori_loop(0, cpt // 2, body, cpt)

    plsc.subcore_barrier()

    for off, sz in ROW_CHUNKS:
        pltpu.sync_copy(agg_sh.at[pl.ds(base + off, sz)], agg_out.at[c, pl.ds(base + off, sz)])


def _sc_aggregate(table, gidx3, didx3, zrow):
    mesh = plsc.VectorSubcoreMesh(core_axis_name="c", subcore_axis_name="s",
                                  num_cores=NC, num_subcores=NS)
    fn = functools.partial(
        pl.kernel,
        out_type=jax.ShapeDtypeStruct((NC, N_PAD, D), jnp.float32),
        mesh=mesh,
        scratch_types=[
            pltpu.VMEM((CPT0, CHUNK), jnp.int32),
            pltpu.VMEM((CPT0, CHUNK), jnp.int32),
            pltpu.VMEM((CHUNK, D), jnp.float32),
            pltpu.VMEM((CHUNK, D), jnp.float32),
            pltpu.VMEM_SHARED((N_PAD, D), jnp.float32),
            pltpu.SemaphoreType.DMA,
            pltpu.SemaphoreType.DMA,
        ],
        compiler_params=pltpu.CompilerParams(use_tc_tiling_on_sc=False),
    )(_sc_agg_body)
    return fn(table, gidx3, didx3, zrow)


def _sc_deg_body(didx_hbm, ones_hbm, z16_hbm,
                 deg_out,
                 didx_v, ones_v,
                 deg_sh):
    c = lax.axis_index("c")
    s = lax.axis_index("s")
    w = c * NS + s
    base = s * ROWS_PT

    pltpu.sync_copy(ones_hbm, ones_v)
    for off, sz in ROW_CHUNKS:
        pltpu.sync_copy(z16_hbm.at[pl.ds(0, sz)], deg_sh.at[pl.ds(base + off, sz)])

    @pl.when(c == 0)
    def _():
        pltpu.sync_copy(didx_hbm.at[pl.ds(NROWS + s * CPT0, CPT0)], didx_v)

    @pl.when(c != 0)
    def _():
        pltpu.sync_copy(didx_hbm.at[pl.ds(NROWS + NS * CPT0 + s * CPT1, CPT1)],
                        didx_v.at[pl.ds(0, CPT1)])

    plsc.subcore_barrier()

    def body(j, carry):
        pltpu.sync_copy(ones_v, deg_sh.at[didx_v.at[j]], add=True)
        return carry

    lax.fori_loop(0, lax.select(c == 0, CPT0, CPT1), body, 0)

    plsc.subcore_barrier()

    for off, sz in ROW_CHUNKS:
        pltpu.sync_copy(deg_sh.at[pl.ds(base + off, sz)], deg_out.at[c, pl.ds(base + off, sz)])


def _sc_degrees(didx3, ones16, z16):
    mesh = plsc.VectorSubcoreMesh(core_axis_name="c", subcore_axis_name="s",
                                  num_cores=NC, num_subcores=NS)
    fn = functools.partial(
        pl.kernel,
        out_type=jax.ShapeDtypeStruct((NC, N_PAD, 16), jnp.float32),
        mesh=mesh,
        scratch_types=[
            pltpu.VMEM((CPT0, CHUNK), jnp.int32),
            pltpu.VMEM((CHUNK, 16), jnp.float32),
            pltpu.VMEM_SHARED((N_PAD, 16), jnp.float32),
        ],
        compiler_params=pltpu.CompilerParams(use_tc_tiling_on_sc=False),
    )(_sc_deg_body)
    return fn(didx3, ones16, z16)


def _h_body(aggp_ref, degp_ref, x_ref, ws_ref, b_ref, h_ref, stats_ref, acc_ref):
    i = pl.program_id(0)

    agg = aggp_ref[0] + aggp_ref[1]
    deg = degp_ref[0, :, 0:1] + degp_ref[1, :, 0:1]
    deg = jnp.maximum(deg, 1.0)
    self_part = jnp.dot(x_ref[...], ws_ref[...], preferred_element_type=jnp.float32)
    h = jnp.maximum(agg / deg + self_part + b_ref[...], 0.0)
    h_ref[...] = h

    rm = jnp.sum(h, axis=1, keepdims=True) * (1.0 / D)
    diff = h - rm
    rv = jnp.sum(diff * diff, axis=1, keepdims=True) * (1.0 / D)
    rs = jnp.sqrt(rv)

    pm = jnp.sum(rm)
    pm2 = jnp.sum(rm * rm)
    ps = jnp.sum(rs)
    ps2 = jnp.sum(rs * rs)

    @pl.when(i == 0)
    def _():
        acc_ref[0] = 0.0
        acc_ref[1] = 0.0
        acc_ref[2] = 0.0
        acc_ref[3] = 0.0

    acc_ref[0] += pm
    acc_ref[1] += pm2
    acc_ref[2] += ps
    acc_ref[3] += ps2

    @pl.when(i == NB - 1)
    def _():
        inv_n = 1.0 / N
        mu_m = acc_ref[0] * inv_n
        var_m = jnp.maximum(acc_ref[1] * inv_n - mu_m * mu_m, 0.0)
        mu_s = acc_ref[2] * inv_n
        var_s = jnp.maximum(acc_ref[3] * inv_n - mu_s * mu_s, 0.0)
        std_mu = jnp.sqrt(var_m)
        std_var = jnp.sqrt(var_s)
        row = lax.broadcasted_iota(jnp.int32, (8, 128), 0)
        stats_ref[...] = jnp.where(row == 0, std_mu, std_var)


def _make_h(aggp, degp, x, W_self, b2):
    return pl.pallas_call(
        _h_body,
        grid=(NB,),
        in_specs=[
            pl.BlockSpec((2, BN, D), lambda i: (0, i, 0)),
            pl.BlockSpec((2, BN, 16), lambda i: (0, i, 0)),
            pl.BlockSpec((BN, D), lambda i: (i, 0)),
            pl.BlockSpec((D, D), lambda i: (0, 0)),
            pl.BlockSpec((1, D), lambda i: (0, 0)),
        ],
        out_specs=[
            pl.BlockSpec((BN, D), lambda i: (i, 0)),
            pl.BlockSpec((8, 128), lambda i: (0, 0)),
        ],
        out_shape=[
            jax.ShapeDtypeStruct((N, D), jnp.float32),
            jax.ShapeDtypeStruct((8, 128), jnp.float32),
        ],
        scratch_shapes=[pltpu.SMEM((4,), jnp.float32)],
    )(aggp, degp, x, W_self, b2)


def _z_body(h_ref, stats_ref, eb_ref, eg_ref, z_ref):
    h = h_ref[...]
    rm = jnp.sum(h, axis=1, keepdims=True) * (1.0 / D)
    diff = h - rm
    rv = jnp.sum(diff * diff, axis=1, keepdims=True) * (1.0 / D)
    rs = jnp.sqrt(rv)
    std_mu = stats_ref[0, 0]
    std_var = stats_ref[1, 0]
    beta = rm + eb_ref[...] * std_mu
    gam = rs + eg_ref[...] * std_var
    z_ref[...] = beta + gam * (diff / (rs + 1e-05))


def _make_z(h, stats, eps_beta, eps_gam):
    return pl.pallas_call(
        _z_body,
        grid=(NB,),
        in_specs=[
            pl.BlockSpec((BN, D), lambda i: (i, 0)),
            pl.BlockSpec((8, 128), lambda i: (0, 0)),
            pl.BlockSpec((BN, 1), lambda i: (i, 0)),
            pl.BlockSpec((BN, 1), lambda i: (i, 0)),
        ],
        out_specs=pl.BlockSpec((BN, D), lambda i: (i, 0)),
        out_shape=jax.ShapeDtypeStruct((N, D), jnp.float32),
    )(h, stats, eps_beta, eps_gam)


def kernel(x, edge_index, edge_type, W_rel, W_self, b, eps_beta, eps_gam):
    eidx128 = edge_index.reshape(2 * (E // 128), 128)
    eidx80 = edge_index.reshape(2 * NROWS, CHUNK)

    table = _make_table(x, W_rel)
    gidx = _make_gidx(eidx128, edge_type.reshape(E // 128, 128))
    gidx3 = gidx.reshape(NROWS, CHUNK)
    didx3 = eidx80

    zrow = jnp.zeros((128, D), jnp.float32)
    ones16 = jnp.ones((CHUNK, 16), jnp.float32)
    z16 = jnp.zeros((128, 16), jnp.float32)

    aggp = _sc_aggregate(table, gidx3, didx3, zrow)
    degp = _sc_degrees(didx3, ones16, z16)

    b2 = b.reshape(1, D)
    h, stats = _make_h(aggp, degp, x, W_self, b2)
    z = _make_z(h, stats, eps_beta, eps_gam)
    return (h, z)


# speedup vs baseline: 1.1570x; 1.0452x over previous
import functools

import jax
import jax.numpy as jnp
from jax import lax
from jax.experimental import pallas as pl
from jax.experimental.pallas import tpu as pltpu
from jax.experimental.pallas import tpu_sc as plsc

N = 10000
E = 320000
D = 128
R = 4

NC = 2
NS = 16
NW = NC * NS

CHUNK = 80
CPT0 = 136
CPT1 = 114
E0 = NS * CPT0 * CHUNK
N_PAD = 10112
ROWS_PT = N_PAD // NS
ROW_CHUNKS = [(0, 128), (128, 128), (256, 128), (384, 128), (512, 120)]

BN = 1000
NB = N // BN
NROWS = E // CHUNK


def _table_body(x_ref, w_ref, o_ref):
    o_ref[...] = jnp.dot(x_ref[...], w_ref[0], preferred_element_type=jnp.float32)


def _make_table(x, W_rel):
    bn = 1000
    nb = N // bn
    return pl.pallas_call(
        _table_body,
        grid=(nb, R),
        in_specs=[
            pl.BlockSpec((bn, D), lambda i, r: (i, 0)),
            pl.BlockSpec((1, D, D), lambda i, r: (r, 0, 0)),
        ],
        out_specs=pl.BlockSpec((bn, D), lambda i, r: (r * nb + i, 0)),
        out_shape=jax.ShapeDtypeStruct((R * N, D), jnp.float32),
    )(x, W_rel)


def _gidx_body(src_ref, et_ref, o_ref):
    o_ref[...] = et_ref[...] * N + src_ref[: E // 128]


def _make_gidx(eidx2d, et2d):
    nrows = E // 128
    return pl.pallas_call(
        _gidx_body,
        grid=(1,),
        in_specs=[
            pl.BlockSpec((2 * nrows, 128), lambda i: (0, 0)),
            pl.BlockSpec((nrows, 128), lambda i: (0, 0)),
        ],
        out_specs=pl.BlockSpec((nrows, 128), lambda i: (0, 0)),
        out_shape=jax.ShapeDtypeStruct((nrows, 128), jnp.int32),
    )(eidx2d, et2d)


def _sc_agg_body(table_hbm, gidx_hbm, didx_hbm, zrow_hbm,
                 agg_out,
                 gidx_v, didx_v, rows0_v, rows1_v,
                 agg_sh, sem0, sem1):
    c = lax.axis_index("c")
    s = lax.axis_index("s")
    w = c * NS + s
    base = s * ROWS_PT

    for off, sz in ROW_CHUNKS:
        pltpu.sync_copy(zrow_hbm.at[pl.ds(0, sz)], agg_sh.at[pl.ds(base + off, sz)])

    @pl.when(c == 0)
    def _():
        row = s * CPT0
        pltpu.sync_copy(gidx_hbm.at[pl.ds(row, CPT0)], gidx_v)
        pltpu.sync_copy(didx_hbm.at[pl.ds(NROWS + row, CPT0)], didx_v)

    @pl.when(c != 0)
    def _():
        row = NS * CPT0 + s * CPT1
        pltpu.sync_copy(gidx_hbm.at[pl.ds(row, CPT1)], gidx_v.at[pl.ds(0, CPT1)])
        pltpu.sync_copy(didx_hbm.at[pl.ds(NROWS + row, CPT1)],
                        didx_v.at[pl.ds(0, CPT1)])

    plsc.subcore_barrier()

    bufs = (rows0_v, rows1_v)
    sems = (sem0, sem1)
    cpt = lax.select(c == 0, CPT0, CPT1)

    pltpu.async_copy(table_hbm.at[gidx_v.at[0]], rows0_v, sem0)
    pltpu.async_copy(table_hbm.at[gidx_v.at[1]], rows1_v, sem1)

    def body(j2, carry):
        for k in range(2):
            j = j2 * 2 + k
            buf, sm = bufs[k], sems[k]
            pltpu.make_async_copy(table_hbm.at[gidx_v.at[j]], buf, sm).wait()
            pltpu.sync_copy(buf, agg_sh.at[didx_v.at[j]], add=True)

            @pl.when(j + 2 < carry)
            def _():
                pltpu.async_copy(table_hbm.at[gidx_v.at[j + 2]], buf, sm)
        return carry

    lax.fori_loop(0, cpt // 2, body, cpt)

    plsc.subcore_barrier()

    for off, sz in ROW_CHUNKS:
        pltpu.sync_copy(agg_sh.at[pl.ds(base + off, sz)], agg_out.at[c, pl.ds(base + off, sz)])


def _sc_aggregate(table, gidx3, didx3, zrow):
    mesh = plsc.VectorSubcoreMesh(core_axis_name="c", subcore_axis_name="s",
                                  num_cores=NC, num_subcores=NS)
    fn = functools.partial(
        pl.kernel,
        out_type=jax.ShapeDtypeStruct((NC, N_PAD, D), jnp.float32),
        mesh=mesh,
        scratch_types=[
            pltpu.VMEM((CPT0, CHUNK), jnp.int32),
            pltpu.VMEM((CPT0, CHUNK), jnp.int32),
            pltpu.VMEM((CHUNK, D), jnp.float32),
            pltpu.VMEM((CHUNK, D), jnp.float32),
            pltpu.VMEM_SHARED((N_PAD, D), jnp.float32),
            pltpu.SemaphoreType.DMA,
            pltpu.SemaphoreType.DMA,
        ],
        compiler_params=pltpu.CompilerParams(use_tc_tiling_on_sc=False),
    )(_sc_agg_body)
    return fn(table, gidx3, didx3, zrow)


def _sc_deg_body(didx_hbm, ones_hbm, z16_hbm,
                 deg_out,
                 didx_v, ones_v,
                 deg_sh):
    c = lax.axis_index("c")
    s = lax.axis_index("s")
    w = c * NS + s
    base = s * ROWS_PT

    pltpu.sync_copy(ones_hbm, ones_v)
    for off, sz in ROW_CHUNKS:
        pltpu.sync_copy(z16_hbm.at[pl.ds(0, sz)], deg_sh.at[pl.ds(base + off, sz)])

    @pl.when(c == 0)
    def _():
        pltpu.sync_copy(didx_hbm.at[pl.ds(NROWS + s * CPT0, CPT0)], didx_v)

    @pl.when(c != 0)
    def _():
        pltpu.sync_copy(didx_hbm.at[pl.ds(NROWS + NS * CPT0 + s * CPT1, CPT1)],
                        didx_v.at[pl.ds(0, CPT1)])

    plsc.subcore_barrier()

    def body(j, carry):
        pltpu.sync_copy(ones_v, deg_sh.at[didx_v.at[j]], add=True)
        return carry

    lax.fori_loop(0, lax.select(c == 0, CPT0, CPT1), body, 0)

    plsc.subcore_barrier()

    for off, sz in ROW_CHUNKS:
        pltpu.sync_copy(deg_sh.at[pl.ds(base + off, sz)], deg_out.at[c, pl.ds(base + off, sz)])


def _sc_degrees(didx3, ones16, z16):
    mesh = plsc.VectorSubcoreMesh(core_axis_name="c", subcore_axis_name="s",
                                  num_cores=NC, num_subcores=NS)
    fn = functools.partial(
        pl.kernel,
        out_type=jax.ShapeDtypeStruct((NC, N_PAD, 16), jnp.float32),
        mesh=mesh,
        scratch_types=[
            pltpu.VMEM((CPT0, CHUNK), jnp.int32),
            pltpu.VMEM((CHUNK, 16), jnp.float32),
            pltpu.VMEM_SHARED((N_PAD, 16), jnp.float32),
        ],
        compiler_params=pltpu.CompilerParams(use_tc_tiling_on_sc=False),
    )(_sc_deg_body)
    return fn(didx3, ones16, z16)


def _h_body(aggp_ref, degp_ref, x_ref, ws_ref, b_ref, h_ref, stats_ref, acc_ref):
    i = pl.program_id(0)

    agg = aggp_ref[0] + aggp_ref[1]
    deg = degp_ref[0, :, 0:1] + degp_ref[1, :, 0:1]
    deg = jnp.maximum(deg, 1.0)
    self_part = jnp.dot(x_ref[...], ws_ref[...], preferred_element_type=jnp.float32)
    h = jnp.maximum(agg / deg + self_part + b_ref[...], 0.0)
    h_ref[...] = h

    rm = jnp.sum(h, axis=1, keepdims=True) * (1.0 / D)
    diff = h - rm
    rv = jnp.sum(diff * diff, axis=1, keepdims=True) * (1.0 / D)
    rs = jnp.sqrt(rv)

    pm = jnp.sum(rm)
    pm2 = jnp.sum(rm * rm)
    ps = jnp.sum(rs)
    ps2 = jnp.sum(rs * rs)

    @pl.when(i == 0)
    def _():
        acc_ref[0] = 0.0
        acc_ref[1] = 0.0
        acc_ref[2] = 0.0
        acc_ref[3] = 0.0

    acc_ref[0] += pm
    acc_ref[1] += pm2
    acc_ref[2] += ps
    acc_ref[3] += ps2

    @pl.when(i == NB - 1)
    def _():
        inv_n = 1.0 / N
        mu_m = acc_ref[0] * inv_n
        var_m = jnp.maximum(acc_ref[1] * inv_n - mu_m * mu_m, 0.0)
        mu_s = acc_ref[2] * inv_n
        var_s = jnp.maximum(acc_ref[3] * inv_n - mu_s * mu_s, 0.0)
        std_mu = jnp.sqrt(var_m)
        std_var = jnp.sqrt(var_s)
        row = lax.broadcasted_iota(jnp.int32, (8, 128), 0)
        stats_ref[...] = jnp.where(row == 0, std_mu, std_var)


def _make_h(aggp, degp, x, W_self, b2):
    return pl.pallas_call(
        _h_body,
        grid=(NB,),
        in_specs=[
            pl.BlockSpec((2, BN, D), lambda i: (0, i, 0)),
            pl.BlockSpec((2, BN, 16), lambda i: (0, i, 0)),
            pl.BlockSpec((BN, D), lambda i: (i, 0)),
            pl.BlockSpec((D, D), lambda i: (0, 0)),
            pl.BlockSpec((1, D), lambda i: (0, 0)),
        ],
        out_specs=[
            pl.BlockSpec((BN, D), lambda i: (i, 0)),
            pl.BlockSpec((8, 128), lambda i: (0, 0)),
        ],
        out_shape=[
            jax.ShapeDtypeStruct((N, D), jnp.float32),
            jax.ShapeDtypeStruct((8, 128), jnp.float32),
        ],
        scratch_shapes=[pltpu.SMEM((4,), jnp.float32)],
    )(aggp, degp, x, W_self, b2)


def _z_body(h_ref, stats_ref, eb_ref, eg_ref, z_ref):
    h = h_ref[...]
    rm = jnp.sum(h, axis=1, keepdims=True) * (1.0 / D)
    diff = h - rm
    rv = jnp.sum(diff * diff, axis=1, keepdims=True) * (1.0 / D)
    rs = jnp.sqrt(rv)
    std_mu = stats_ref[0, 0]
    std_var = stats_ref[1, 0]
    beta = rm + eb_ref[...] * std_mu
    gam = rs + eg_ref[...] * std_var
    z_ref[...] = beta + gam * (diff / (rs + 1e-05))


def _make_z(h, stats, eps_beta, eps_gam):
    return pl.pallas_call(
        _z_body,
        grid=(NB,),
        in_specs=[
            pl.BlockSpec((BN, D), lambda i: (i, 0)),
            pl.BlockSpec((8, 128), lambda i: (0, 0)),
            pl.BlockSpec((BN, 1), lambda i: (i, 0)),
            pl.BlockSpec((BN, 1), lambda i: (i, 0)),
        ],
        out_specs=pl.BlockSpec((BN, D), lambda i: (i, 0)),
        out_shape=jax.ShapeDtypeStruct((N, D), jnp.float32),
    )(h, stats, eps_beta, eps_gam)


def kernel(x, edge_index, edge_type, W_rel, W_self, b, eps_beta, eps_gam):
    eidx128 = edge_index.reshape(2 * (E // 128), 128)
    eidx80 = edge_index.reshape(2 * NROWS, CHUNK)

    table = _make_table(x, W_rel)
    gidx = _make_gidx(eidx128, edge_type.reshape(E // 128, 128))
    gidx3 = gidx.reshape(NROWS, CHUNK)
    didx3 = eidx80

    zrow = jnp.zeros((128, D), jnp.float32)
    ones16 = jnp.ones((CHUNK, 16), jnp.float32)
    z16 = jnp.zeros((128, 16), jnp.float32)

    aggp = _sc_aggregate(table, gidx3, didx3, zrow)
    degp = _sc_degrees(didx3, ones16, z16)

    b2 = b.reshape(1, D)
    h, stats = _make_h(aggp, degp, x, W_self, b2)
    z = _make_z(h, stats, eps_beta, eps_gam)
    return (h, z)


# speedup vs baseline: 1.1953x; 1.0331x over previous
import functools

import jax
import jax.numpy as jnp
from jax import lax
from jax.experimental import pallas as pl
from jax.experimental.pallas import tpu as pltpu
from jax.experimental.pallas import tpu_sc as plsc

N = 10000
E = 320000
D = 128
R = 4

NC = 2
NS = 16
NW = NC * NS

CHUNK = 80
CPT0 = 126
CPT1 = 124
E0 = NS * CPT0 * CHUNK
N_PAD = 10112
ROWS_PT = N_PAD // NS
ROW_CHUNKS = [(0, 128), (128, 128), (256, 128), (384, 128), (512, 120)]

BN = 1000
NB = N // BN
NROWS = E // CHUNK


def _table_body(x_ref, w_ref, o_ref):
    o_ref[...] = jnp.dot(x_ref[...], w_ref[0], preferred_element_type=jnp.float32)


def _make_table(x, W_rel):
    bn = 1000
    nb = N // bn
    return pl.pallas_call(
        _table_body,
        grid=(nb, R),
        in_specs=[
            pl.BlockSpec((bn, D), lambda i, r: (i, 0)),
            pl.BlockSpec((1, D, D), lambda i, r: (r, 0, 0)),
        ],
        out_specs=pl.BlockSpec((bn, D), lambda i, r: (r * nb + i, 0)),
        out_shape=jax.ShapeDtypeStruct((R * N, D), jnp.float32),
    )(x, W_rel)


def _gidx_body(src_ref, et_ref, o_ref):
    o_ref[...] = et_ref[...] * N + src_ref[: E // 128]


def _make_gidx(eidx2d, et2d):
    nrows = E // 128
    return pl.pallas_call(
        _gidx_body,
        grid=(1,),
        in_specs=[
            pl.BlockSpec((2 * nrows, 128), lambda i: (0, 0)),
            pl.BlockSpec((nrows, 128), lambda i: (0, 0)),
        ],
        out_specs=pl.BlockSpec((nrows, 128), lambda i: (0, 0)),
        out_shape=jax.ShapeDtypeStruct((nrows, 128), jnp.int32),
    )(eidx2d, et2d)


def _sc_agg_body(table_hbm, gidx_hbm, didx_hbm, zrow_hbm,
                 agg_out,
                 gidx_v, didx_v, rows0_v, rows1_v,
                 agg_sh, sem0, sem1):
    c = lax.axis_index("c")
    s = lax.axis_index("s")
    w = c * NS + s
    base = s * ROWS_PT

    for off, sz in ROW_CHUNKS:
        pltpu.sync_copy(zrow_hbm.at[pl.ds(0, sz)], agg_sh.at[pl.ds(base + off, sz)])

    @pl.when(c == 0)
    def _():
        row = s * CPT0
        pltpu.sync_copy(gidx_hbm.at[pl.ds(row, CPT0)], gidx_v)
        pltpu.sync_copy(didx_hbm.at[pl.ds(NROWS + row, CPT0)], didx_v)

    @pl.when(c != 0)
    def _():
        row = NS * CPT0 + s * CPT1
        pltpu.sync_copy(gidx_hbm.at[pl.ds(row, CPT1)], gidx_v.at[pl.ds(0, CPT1)])
        pltpu.sync_copy(didx_hbm.at[pl.ds(NROWS + row, CPT1)],
                        didx_v.at[pl.ds(0, CPT1)])

    plsc.subcore_barrier()

    bufs = (rows0_v, rows1_v)
    sems = (sem0, sem1)
    cpt = lax.select(c == 0, CPT0, CPT1)

    pltpu.async_copy(table_hbm.at[gidx_v.at[0]], rows0_v, sem0)
    pltpu.async_copy(table_hbm.at[gidx_v.at[1]], rows1_v, sem1)

    def body(j2, carry):
        for k in range(2):
            j = j2 * 2 + k
            buf, sm = bufs[k], sems[k]
            pltpu.make_async_copy(table_hbm.at[gidx_v.at[j]], buf, sm).wait()
            pltpu.sync_copy(buf, agg_sh.at[didx_v.at[j]], add=True)

            @pl.when(j + 2 < carry)
            def _():
                pltpu.async_copy(table_hbm.at[gidx_v.at[j + 2]], buf, sm)
        return carry

    lax.fori_loop(0, cpt // 2, body, cpt)

    plsc.subcore_barrier()

    for off, sz in ROW_CHUNKS:
        pltpu.sync_copy(agg_sh.at[pl.ds(base + off, sz)], agg_out.at[c, pl.ds(base + off, sz)])


def _sc_aggregate(table, gidx3, didx3, zrow):
    mesh = plsc.VectorSubcoreMesh(core_axis_name="c", subcore_axis_name="s",
                                  num_cores=NC, num_subcores=NS)
    fn = functools.partial(
        pl.kernel,
        out_type=jax.ShapeDtypeStruct((NC, N_PAD, D), jnp.float32),
        mesh=mesh,
        scratch_types=[
            pltpu.VMEM((CPT0, CHUNK), jnp.int32),
            pltpu.VMEM((CPT0, CHUNK), jnp.int32),
            pltpu.VMEM((CHUNK, D), jnp.float32),
            pltpu.VMEM((CHUNK, D), jnp.float32),
            pltpu.VMEM_SHARED((N_PAD, D), jnp.float32),
            pltpu.SemaphoreType.DMA,
            pltpu.SemaphoreType.DMA,
        ],
        compiler_params=pltpu.CompilerParams(use_tc_tiling_on_sc=False),
    )(_sc_agg_body)
    return fn(table, gidx3, didx3, zrow)


def _sc_deg_body(didx_hbm, ones_hbm, z16_hbm,
                 deg_out,
                 didx_v, ones_v,
                 deg_sh):
    c = lax.axis_index("c")
    s = lax.axis_index("s")
    w = c * NS + s
    base = s * ROWS_PT

    pltpu.sync_copy(ones_hbm, ones_v)
    for off, sz in ROW_CHUNKS:
        pltpu.sync_copy(z16_hbm.at[pl.ds(0, sz)], deg_sh.at[pl.ds(base + off, sz)])

    @pl.when(c == 0)
    def _():
        pltpu.sync_copy(didx_hbm.at[pl.ds(NROWS + s * CPT0, CPT0)], didx_v)

    @pl.when(c != 0)
    def _():
        pltpu.sync_copy(didx_hbm.at[pl.ds(NROWS + NS * CPT0 + s * CPT1, CPT1)],
                        didx_v.at[pl.ds(0, CPT1)])

    plsc.subcore_barrier()

    def body(j, carry):
        pltpu.sync_copy(ones_v, deg_sh.at[didx_v.at[j]], add=True)
        return carry

    lax.fori_loop(0, lax.select(c == 0, CPT0, CPT1), body, 0)

    plsc.subcore_barrier()

    for off, sz in ROW_CHUNKS:
        pltpu.sync_copy(deg_sh.at[pl.ds(base + off, sz)], deg_out.at[c, pl.ds(base + off, sz)])


def _sc_degrees(didx3, ones16, z16):
    mesh = plsc.VectorSubcoreMesh(core_axis_name="c", subcore_axis_name="s",
                                  num_cores=NC, num_subcores=NS)
    fn = functools.partial(
        pl.kernel,
        out_type=jax.ShapeDtypeStruct((NC, N_PAD, 16), jnp.float32),
        mesh=mesh,
        scratch_types=[
            pltpu.VMEM((CPT0, CHUNK), jnp.int32),
            pltpu.VMEM((CHUNK, 16), jnp.float32),
            pltpu.VMEM_SHARED((N_PAD, 16), jnp.float32),
        ],
        compiler_params=pltpu.CompilerParams(use_tc_tiling_on_sc=False),
    )(_sc_deg_body)
    return fn(didx3, ones16, z16)


def _h_body(aggp_ref, degp_ref, x_ref, ws_ref, b_ref, h_ref, stats_ref, acc_ref):
    i = pl.program_id(0)

    agg = aggp_ref[0] + aggp_ref[1]
    deg = degp_ref[0, :, 0:1] + degp_ref[1, :, 0:1]
    deg = jnp.maximum(deg, 1.0)
    self_part = jnp.dot(x_ref[...], ws_ref[...], preferred_element_type=jnp.float32)
    h = jnp.maximum(agg / deg + self_part + b_ref[...], 0.0)
    h_ref[...] = h

    rm = jnp.sum(h, axis=1, keepdims=True) * (1.0 / D)
    diff = h - rm
    rv = jnp.sum(diff * diff, axis=1, keepdims=True) * (1.0 / D)
    rs = jnp.sqrt(rv)

    pm = jnp.sum(rm)
    pm2 = jnp.sum(rm * rm)
    ps = jnp.sum(rs)
    ps2 = jnp.sum(rs * rs)

    @pl.when(i == 0)
    def _():
        acc_ref[0] = 0.0
        acc_ref[1] = 0.0
        acc_ref[2] = 0.0
        acc_ref[3] = 0.0

    acc_ref[0] += pm
    acc_ref[1] += pm2
    acc_ref[2] += ps
    acc_ref[3] += ps2

    @pl.when(i == NB - 1)
    def _():
        inv_n = 1.0 / N
        mu_m = acc_ref[0] * inv_n
        var_m = jnp.maximum(acc_ref[1] * inv_n - mu_m * mu_m, 0.0)
        mu_s = acc_ref[2] * inv_n
        var_s = jnp.maximum(acc_ref[3] * inv_n - mu_s * mu_s, 0.0)
        std_mu = jnp.sqrt(var_m)
        std_var = jnp.sqrt(var_s)
        row = lax.broadcasted_iota(jnp.int32, (8, 128), 0)
        stats_ref[...] = jnp.where(row == 0, std_mu, std_var)


def _make_h(aggp, degp, x, W_self, b2):
    return pl.pallas_call(
        _h_body,
        grid=(NB,),
        in_specs=[
            pl.BlockSpec((2, BN, D), lambda i: (0, i, 0)),
            pl.BlockSpec((2, BN, 16), lambda i: (0, i, 0)),
            pl.BlockSpec((BN, D), lambda i: (i, 0)),
            pl.BlockSpec((D, D), lambda i: (0, 0)),
            pl.BlockSpec((1, D), lambda i: (0, 0)),
        ],
        out_specs=[
            pl.BlockSpec((BN, D), lambda i: (i, 0)),
            pl.BlockSpec((8, 128), lambda i: (0, 0)),
        ],
        out_shape=[
            jax.ShapeDtypeStruct((N, D), jnp.float32),
            jax.ShapeDtypeStruct((8, 128), jnp.float32),
        ],
        scratch_shapes=[pltpu.SMEM((4,), jnp.float32)],
    )(aggp, degp, x, W_self, b2)


def _z_body(h_ref, stats_ref, eb_ref, eg_ref, z_ref):
    h = h_ref[...]
    rm = jnp.sum(h, axis=1, keepdims=True) * (1.0 / D)
    diff = h - rm
    rv = jnp.sum(diff * diff, axis=1, keepdims=True) * (1.0 / D)
    rs = jnp.sqrt(rv)
    std_mu = stats_ref[0, 0]
    std_var = stats_ref[1, 0]
    beta = rm + eb_ref[...] * std_mu
    gam = rs + eg_ref[...] * std_var
    z_ref[...] = beta + gam * (diff / (rs + 1e-05))


def _make_z(h, stats, eps_beta, eps_gam):
    return pl.pallas_call(
        _z_body,
        grid=(NB,),
        in_specs=[
            pl.BlockSpec((BN, D), lambda i: (i, 0)),
            pl.BlockSpec((8, 128), lambda i: (0, 0)),
            pl.BlockSpec((BN, 1), lambda i: (i, 0)),
            pl.BlockSpec((BN, 1), lambda i: (i, 0)),
        ],
        out_specs=pl.BlockSpec((BN, D), lambda i: (i, 0)),
        out_shape=jax.ShapeDtypeStruct((N, D), jnp.float32),
    )(h, stats, eps_beta, eps_gam)


def kernel(x, edge_index, edge_type, W_rel, W_self, b, eps_beta, eps_gam):
    eidx128 = edge_index.reshape(2 * (E // 128), 128)
    eidx80 = edge_index.reshape(2 * NROWS, CHUNK)

    table = _make_table(x, W_rel)
    gidx = _make_gidx(eidx128, edge_type.reshape(E // 128, 128))
    gidx3 = gidx.reshape(NROWS, CHUNK)
    didx3 = eidx80

    zrow = jnp.zeros((128, D), jnp.float32)
    ones16 = jnp.ones((CHUNK, 16), jnp.float32)
    z16 = jnp.zeros((128, 16), jnp.float32)

    aggp = _sc_aggregate(table, gidx3, didx3, zrow)
    degp = _sc_degrees(didx3, ones16, z16)

    b2 = b.reshape(1, D)
    h, stats = _make_h(aggp, degp, x, W_self, b2)
    z = _make_z(h, stats, eps_beta, eps_gam)
    return (h, z)
